# baseline (speedup 1.0000x reference)
import jax
import jax.numpy as jnp
from jax import lax
from jax.experimental import pallas as pl
from jax.experimental.pallas import tpu as pltpu
from jax.experimental.pallas import tpu_sc as plsc

D = 64
SCALE = float(D) ** 0.5
NC = 2
NS = 16
NW = NC * NS
SPB = 128
CH = 256
NSTREAM = CH // SPB
NBUF = 4
PF = 2


def _make_kernel(B):
    n_chunks = B // CH // NW
    assert n_chunks % NBUF == 0 and n_chunks >= 2 * NBUF

    def body(idx_hbm, table_hbm, out_hbm, idx_all, *bufs):
        rows = bufs[:NBUF]
        gsem = bufs[NBUF:2 * NBUF]
        ssem = bufs[2 * NBUF:3 * NBUF]
        wid = lax.axis_index("s") * NC + lax.axis_index("c")
        base_row = wid * (n_chunks * NSTREAM)
        base = base_row * SPB

        def fire_gather(g, b):
            for j in range(NSTREAM):
                pltpu.async_copy(
                    table_hbm.at[idx_all.at[g * NSTREAM + j]],
                    rows[b].at[pl.ds(j * SPB, SPB)],
                    gsem[b],
                )

        def wait_gather(b):
            for j in range(NSTREAM):
                pltpu.make_async_copy(
                    table_hbm.at[idx_all.at[0]],
                    rows[b].at[pl.ds(j * SPB, SPB)],
                    gsem[b],
                ).wait()

        def fire_store(g, b):
            pltpu.async_copy(rows[b], out_hbm.at[pl.ds(base + g * CH, CH)],
                             ssem[b])

        def wait_store(b):
            pltpu.make_async_copy(rows[b], out_hbm.at[pl.ds(base, CH)],
                                  ssem[b]).wait()

        pltpu.sync_copy(idx_hbm.at[pl.ds(base_row, n_chunks * NSTREAM)],
                        idx_all)
        for g in range(PF):
            fire_gather(g, g)

        def ring(go, _):
            for s in range(NBUF):
                g = go + s
                b = s

                @pl.when(g >= NBUF - PF)
                def _():
                    wait_store((b + PF) % NBUF)

                @pl.when(g + PF < n_chunks)
                def _():
                    fire_gather(g + PF, (b + PF) % NBUF)

                wait_gather(b)

                @plsc.parallel_loop(0, CH, 1, unroll=8)
                def _(i):
                    for k in range(D // 16):
                        sl = pl.ds(k * 16, 16)
                        rows[b][i, sl] = rows[b][i, sl] * SCALE

                fire_store(g, b)
            return ()

        lax.fori_loop(0, n_chunks // NBUF, lambda q, c: ring(q * NBUF, c), ())
        for g in range(n_chunks - PF, n_chunks):
            wait_store(g % NBUF)

    mesh = plsc.VectorSubcoreMesh(
        core_axis_name="c", subcore_axis_name="s", num_cores=NC, num_subcores=NS
    )
    return pl.kernel(
        body,
        out_type=jax.ShapeDtypeStruct((B, D), jnp.float32),
        mesh=mesh,
        scratch_types=(
            [pltpu.VMEM((B // SPB // NW, SPB), jnp.int32)]
            + [pltpu.VMEM((CH, D), jnp.float32)] * NBUF
            + [pltpu.SemaphoreType.DMA] * (2 * NBUF)
        ),
        compiler_params=pltpu.CompilerParams(use_tc_tiling_on_sc=False),
    )


def kernel(x, embedding_weight):
    B = x.shape[0] * x.shape[1]
    idx = x.reshape(B // SPB, SPB).astype(jnp.int32)
    out = _make_kernel(B)(idx, embedding_weight)
    return out.reshape(x.shape[0], x.shape[1], D)


# speedup vs baseline: 1.0012x; 1.0012x over previous
import jax
import jax.numpy as jnp
from jax import lax
from jax.experimental import pallas as pl
from jax.experimental.pallas import tpu as pltpu
from jax.experimental.pallas import tpu_sc as plsc

D = 64
SCALE = float(D) ** 0.5
NC = 2
NS = 16
NW = NC * NS
SPB = 128
CH = 256
NSTREAM = CH // SPB
NBUF = 4
PF = 2


def _make_kernel(B):
    n_chunks = B // CH // NW
    assert n_chunks % NBUF == 0 and n_chunks >= 2 * NBUF

    def body(idx_hbm, table_hbm, out_hbm, idx_all, *bufs):
        rows = bufs[:NBUF]
        gsem = bufs[NBUF:2 * NBUF]
        ssem = bufs[2 * NBUF:3 * NBUF]
        wid = lax.axis_index("s") * NC + lax.axis_index("c")
        base_row = wid * (n_chunks * NSTREAM)
        base = base_row * SPB

        def fire_gather(g, b):
            for j in range(NSTREAM):
                pltpu.async_copy(
                    table_hbm.at[idx_all.at[g * NSTREAM + j]],
                    rows[b].at[pl.ds(j * SPB, SPB)],
                    gsem[b],
                )

        def wait_gather(b):
            for j in range(NSTREAM):
                pltpu.make_async_copy(
                    table_hbm.at[idx_all.at[0]],
                    rows[b].at[pl.ds(j * SPB, SPB)],
                    gsem[b],
                ).wait()

        def fire_store(g, b):
            pltpu.async_copy(rows[b], out_hbm.at[pl.ds(base + g * CH, CH)],
                             ssem[b])

        def wait_store(b):
            pltpu.make_async_copy(rows[b], out_hbm.at[pl.ds(base, CH)],
                                  ssem[b]).wait()

        pltpu.sync_copy(idx_hbm.at[pl.ds(base_row, n_chunks * NSTREAM)],
                        idx_all)
        for g in range(PF):
            fire_gather(g, g)

        def ring(go, _):
            for s in range(NBUF):
                g = go + s
                b = s

                @pl.when(g >= NBUF - PF)
                def _():
                    wait_store((b + PF) % NBUF)

                @pl.when(g + PF < n_chunks)
                def _():
                    fire_gather(g + PF, (b + PF) % NBUF)

                for j in range(NSTREAM):
                    pltpu.make_async_copy(
                        table_hbm.at[idx_all.at[0]],
                        rows[b].at[pl.ds(j * SPB, SPB)],
                        gsem[b],
                    ).wait()

                    @plsc.parallel_loop(0, SPB, 1, unroll=8)
                    def _(i):
                        for k in range(D // 16):
                            sl = pl.ds(k * 16, 16)
                            r = i + j * SPB
                            rows[b][r, sl] = rows[b][r, sl] * SCALE

                fire_store(g, b)
            return ()

        lax.fori_loop(0, n_chunks // NBUF, lambda q, c: ring(q * NBUF, c), ())
        for g in range(n_chunks - PF, n_chunks):
            wait_store(g % NBUF)

    mesh = plsc.VectorSubcoreMesh(
        core_axis_name="c", subcore_axis_name="s", num_cores=NC, num_subcores=NS
    )
    return pl.kernel(
        body,
        out_type=jax.ShapeDtypeStruct((B, D), jnp.float32),
        mesh=mesh,
        scratch_types=(
            [pltpu.VMEM((B // SPB // NW, SPB), jnp.int32)]
            + [pltpu.VMEM((CH, D), jnp.float32)] * NBUF
            + [pltpu.SemaphoreType.DMA] * (2 * NBUF)
        ),
        compiler_params=pltpu.CompilerParams(use_tc_tiling_on_sc=False),
    )


def kernel(x, embedding_weight):
    B = x.shape[0] * x.shape[1]
    idx = x.reshape(B // SPB, SPB).astype(jnp.int32)
    out = _make_kernel(B)(idx, embedding_weight)
    return out.reshape(x.shape[0], x.shape[1], D)
